# baseline (speedup 1.0000x reference)
import numpy as np
import jax
import jax.numpy as jnp
from jax.experimental import pallas as pl
from jax.experimental.pallas import tpu as pltpu

_NUM_CLASSES = 80
_STRIDE = 16
_SCALES = [1.0]
_ASPECTS = [0.5, 1.0, 2.0]
_FH, _FW = 80, 80
_ENC = 0.1


def _anchors_np():
    scales = np.array(_SCALES, dtype=np.float32) * _STRIDE
    aspects = np.array(_ASPECTS, dtype=np.float32)
    sizes = scales[:, None] * np.array([1.0, 1.0], dtype=np.float32)[None, :]
    ratios = np.stack([np.sqrt(aspects), 1.0 / np.sqrt(aspects)], axis=-1)
    sizes = (ratios[None, ...] * sizes[:, None, :]).reshape(-1, 2)
    layout = np.concatenate([np.zeros_like(sizes), sizes], axis=-1)
    vx = (np.arange(_FW, dtype=np.float32) + 0.5) * _STRIDE
    vy = (np.arange(_FH, dtype=np.float32) + 0.5) * _STRIDE
    vyg, vxg = np.meshgrid(vy, vx, indexing="ij")
    offsets = np.stack([vxg, vyg], axis=-1)
    anchors = np.tile(layout[None, None, :, :], (_FH, _FW, 1, 1))
    anchors[:, :, :, :2] += offsets[:, :, None, :]
    return anchors.reshape(_FW, 240, 4)


_C = 3 * _NUM_CLASSES
_CB = 24
_BC = 12


_HSTEPS = 5
_HB = _FH // _HSTEPS


def _body(s_ref, b_ref, a_ref, so_ref, bo_ref):
    i = pl.program_id(0)
    xs = s_ref[0, :, pl.ds(i * _HB, _HB), :]
    y = jax.nn.sigmoid(jnp.transpose(xs, (1, 2, 0)))
    for w in range(_FW):
        so_ref[:, 240 * w:240 * (w + 1)] = y[:, w, :]

    @pl.when(i == _HSTEPS - 1)
    def _decode():
        _bbox(b_ref, a_ref, bo_ref)


def _bbox(b_ref, a_ref, bo_ref):
    x = b_ref[0].reshape(_BC, 20, 4, _FW)

    def plane(j):
        return jnp.transpose(x[:, :, j, :].reshape(_BC * 20, _FW))

    t0 = plane(0) * _ENC + _ENC
    t1 = plane(1) * _ENC + _ENC
    t2 = plane(2) * (2 * _ENC) + 2 * _ENC
    t3 = plane(3) * (2 * _ENC) + 2 * _ENC
    a0, a1, a2, a3 = (a_ref[j] for j in range(4))
    cx = t0 * a2 + a0
    cy = t1 * a3 + a1
    hw = 0.5 * jnp.exp(t2) * a2
    hh = 0.5 * jnp.exp(t3) * a3
    planes = (cx - hw, cy - hh, cx + hw, cy + hh)
    for j, p in enumerate(planes):
        for w in range(_FW):
            bo_ref[j, 240 * w:240 * (w + 1)] = p[w, :]


def kernel(score, bbox):
    anch = jnp.asarray(_anchors_np().transpose(2, 0, 1))
    so, bo = pl.pallas_call(
        _body,
        grid=(_HSTEPS,),
        in_specs=[
            pl.BlockSpec(memory_space=pltpu.MemorySpace.VMEM),
            pl.BlockSpec((1, _BC, _FH, _FW), lambda i: (0, 0, 0, 0)),
            pl.BlockSpec((4, _FW, 240), lambda i: (0, 0, 0)),
        ],
        out_specs=[
            pl.BlockSpec((_HB, _FW * _C), lambda i: (i, 0)),
            pl.BlockSpec((4, _FW * 240), lambda i: (0, 0)),
        ],
        out_shape=[
            jax.ShapeDtypeStruct((_FH, _FW * _C), jnp.float32),
            jax.ShapeDtypeStruct((4, _FW * 240), jnp.float32),
        ],
    )(score, bbox, anch)
    return jnp.transpose(so), jnp.transpose(bo)


# speedup vs baseline: 1.0435x; 1.0435x over previous
import numpy as np
import jax
import jax.numpy as jnp
from jax.experimental import pallas as pl
from jax.experimental.pallas import tpu as pltpu

_NUM_CLASSES = 80
_STRIDE = 16
_SCALES = [1.0]
_ASPECTS = [0.5, 1.0, 2.0]
_FH, _FW = 80, 80
_ENC = 0.1


def _anchors_np():
    scales = np.array(_SCALES, dtype=np.float32) * _STRIDE
    aspects = np.array(_ASPECTS, dtype=np.float32)
    sizes = scales[:, None] * np.array([1.0, 1.0], dtype=np.float32)[None, :]
    ratios = np.stack([np.sqrt(aspects), 1.0 / np.sqrt(aspects)], axis=-1)
    sizes = (ratios[None, ...] * sizes[:, None, :]).reshape(-1, 2)
    layout = np.concatenate([np.zeros_like(sizes), sizes], axis=-1)
    vx = (np.arange(_FW, dtype=np.float32) + 0.5) * _STRIDE
    vy = (np.arange(_FH, dtype=np.float32) + 0.5) * _STRIDE
    vyg, vxg = np.meshgrid(vy, vx, indexing="ij")
    offsets = np.stack([vxg, vyg], axis=-1)
    anchors = np.tile(layout[None, None, :, :], (_FH, _FW, 1, 1))
    anchors[:, :, :, :2] += offsets[:, :, None, :]
    return anchors.reshape(_FW, 240, 4)


_C = 3 * _NUM_CLASSES
_CB = 24
_BC = 12


_HSTEPS = 5
_HB = _FH // _HSTEPS


def _body(s_ref, b_ref, a_ref, so_ref, bo_ref):
    i = pl.program_id(0)
    y = jax.nn.sigmoid(jnp.transpose(s_ref[0], (1, 2, 0)))
    for w in range(_FW):
        so_ref[:, 240 * w:240 * (w + 1)] = y[:, w, :]

    @pl.when(i == _HSTEPS - 1)
    def _decode():
        _bbox(b_ref, a_ref, bo_ref)


def _bbox(b_ref, a_ref, bo_ref):
    x = b_ref[0].reshape(_BC, 20, 4, _FW)

    def plane(j):
        return jnp.transpose(x[:, :, j, :].reshape(_BC * 20, _FW))

    t0 = plane(0) * _ENC + _ENC
    t1 = plane(1) * _ENC + _ENC
    t2 = plane(2) * (2 * _ENC) + 2 * _ENC
    t3 = plane(3) * (2 * _ENC) + 2 * _ENC
    a0, a1, a2, a3 = (a_ref[j] for j in range(4))
    cx = t0 * a2 + a0
    cy = t1 * a3 + a1
    hw = 0.5 * jnp.exp(t2) * a2
    hh = 0.5 * jnp.exp(t3) * a3
    planes = (cx - hw, cy - hh, cx + hw, cy + hh)
    for j, p in enumerate(planes):
        for w in range(_FW):
            bo_ref[j, 240 * w:240 * (w + 1)] = p[w, :]


def kernel(score, bbox):
    anch = jnp.asarray(_anchors_np().transpose(2, 0, 1))
    so, bo = pl.pallas_call(
        _body,
        grid=(_HSTEPS,),
        in_specs=[
            pl.BlockSpec((1, _C, _HB, _FW), lambda i: (0, 0, i, 0)),
            pl.BlockSpec((1, _BC, _FH, _FW), lambda i: (0, 0, 0, 0)),
            pl.BlockSpec((4, _FW, 240), lambda i: (0, 0, 0)),
        ],
        out_specs=[
            pl.BlockSpec((_HB, _FW * _C), lambda i: (i, 0)),
            pl.BlockSpec((4, _FW * 240), lambda i: (0, 0)),
        ],
        out_shape=[
            jax.ShapeDtypeStruct((_FH, _FW * _C), jnp.float32),
            jax.ShapeDtypeStruct((4, _FW * 240), jnp.float32),
        ],
    )(score, bbox, anch)
    return jnp.transpose(so), jnp.transpose(bo)


# speedup vs baseline: 1.6889x; 1.6186x over previous
import numpy as np
import jax
import jax.numpy as jnp
from jax.experimental import pallas as pl
from jax.experimental.pallas import tpu as pltpu

_NUM_CLASSES = 80
_STRIDE = 16
_SCALES = [1.0]
_ASPECTS = [0.5, 1.0, 2.0]
_FH, _FW = 80, 80
_ENC = 0.1


def _anchors_np():
    scales = np.array(_SCALES, dtype=np.float32) * _STRIDE
    aspects = np.array(_ASPECTS, dtype=np.float32)
    sizes = scales[:, None] * np.array([1.0, 1.0], dtype=np.float32)[None, :]
    ratios = np.stack([np.sqrt(aspects), 1.0 / np.sqrt(aspects)], axis=-1)
    sizes = (ratios[None, ...] * sizes[:, None, :]).reshape(-1, 2)
    layout = np.concatenate([np.zeros_like(sizes), sizes], axis=-1)
    vx = (np.arange(_FW, dtype=np.float32) + 0.5) * _STRIDE
    vy = (np.arange(_FH, dtype=np.float32) + 0.5) * _STRIDE
    vyg, vxg = np.meshgrid(vy, vx, indexing="ij")
    offsets = np.stack([vxg, vyg], axis=-1)
    anchors = np.tile(layout[None, None, :, :], (_FH, _FW, 1, 1))
    anchors[:, :, :, :2] += offsets[:, :, None, :]
    return anchors.reshape(_FW, 240, 4)


_C = 3 * _NUM_CLASSES
_CB = 24
_BC = 12


def _body(s_ref, b_ref, a_ref, so_ref, bo_ref):
    y = jax.nn.sigmoid(jnp.transpose(s_ref[0], (1, 2, 0)))
    for w in range(_FW):
        so_ref[:, 240 * w:240 * (w + 1)] = y[:, w, :]
    _bbox(b_ref, a_ref, bo_ref)


def _bbox(b_ref, a_ref, bo_ref):
    x = b_ref[0].reshape(_BC, 20, 4, _FW)

    def plane(j):
        return jnp.transpose(x[:, :, j, :].reshape(_BC * 20, _FW))

    t0 = plane(0) * _ENC + _ENC
    t1 = plane(1) * _ENC + _ENC
    t2 = plane(2) * (2 * _ENC) + 2 * _ENC
    t3 = plane(3) * (2 * _ENC) + 2 * _ENC
    a0, a1, a2, a3 = (a_ref[j] for j in range(4))
    cx = t0 * a2 + a0
    cy = t1 * a3 + a1
    hw = 0.5 * jnp.exp(t2) * a2
    hh = 0.5 * jnp.exp(t3) * a3
    planes = (cx - hw, cy - hh, cx + hw, cy + hh)
    for j, p in enumerate(planes):
        for w in range(_FW):
            bo_ref[j, 240 * w:240 * (w + 1)] = p[w, :]


def kernel(score, bbox):
    anch = jnp.asarray(_anchors_np().transpose(2, 0, 1))
    so, bo = pl.pallas_call(
        _body,
        in_specs=[
            pl.BlockSpec((1, _C, _FH, _FW), lambda: (0, 0, 0, 0)),
            pl.BlockSpec((1, _BC, _FH, _FW), lambda: (0, 0, 0, 0)),
            pl.BlockSpec((4, _FW, 240), lambda: (0, 0, 0)),
        ],
        out_specs=[
            pl.BlockSpec((_FH, _FW * _C), lambda: (0, 0)),
            pl.BlockSpec((4, _FW * 240), lambda: (0, 0)),
        ],
        out_shape=[
            jax.ShapeDtypeStruct((_FH, _FW * _C), jnp.float32),
            jax.ShapeDtypeStruct((4, _FW * 240), jnp.float32),
        ],
    )(score, bbox, anch)
    return jnp.transpose(so), jnp.transpose(bo)
